# Initial kernel scaffold; baseline (speedup 1.0000x reference)
import jax
import jax.numpy as jnp
from jax.experimental import pallas as pl


def kernel(inputs, table):
    raise NotImplementedError("write your pallas kernel here")


# speedup vs baseline: 2.7565x; 2.7565x over previous
import functools

import jax
import jax.numpy as jnp
from jax import lax
from jax.experimental import pallas as pl
from jax.experimental.pallas import tpu as pltpu
from jax.experimental.pallas import tpu_sc as plsc

NC, NS = 2, 16
NW = NC * NS
B, L, D = 16384, 50, 32
BPW = B // NW
BR = 16
NBLK = BPW // BR
IDX_PER_W = BPW * L
IDX_PER_BLK = BR * L
CH = 80
NCH = IDX_PER_BLK // CH
HALF = 16
INV_L = 1.0 / L

_mesh = plsc.VectorSubcoreMesh(
    core_axis_name="c", subcore_axis_name="s", num_cores=NC, num_subcores=NS
)


@functools.partial(
    pl.kernel,
    out_type=jax.ShapeDtypeStruct((B, D), jnp.float32),
    mesh=_mesh,
    scratch_types=[
        pltpu.VMEM((IDX_PER_W,), jnp.int32),
        pltpu.VMEM((IDX_PER_BLK, D), jnp.float32),
        pltpu.VMEM((BPW, D), jnp.float32),
        pltpu.SemaphoreType.DMA,
    ],
    compiler_params=pltpu.CompilerParams(use_tc_tiling_on_sc=False),
)
def _emb_lookup_mean(table_hbm, idx_hbm, out_hbm, idx_v, rows_v, out_v, sem):
    wid = lax.axis_index("s") * NC + lax.axis_index("c")
    pltpu.sync_copy(idx_hbm.at[pl.ds(wid * IDX_PER_W, IDX_PER_W)], idx_v)

    def block(g, carry):
        base = pl.multiple_of(g * IDX_PER_BLK, 8)
        copies = []
        for c in range(NCH):
            copies.append(
                pltpu.async_copy(
                    table_hbm.at[idx_v.at[pl.ds(base + c * CH, CH)]],
                    rows_v.at[pl.ds(c * CH, CH)],
                    sem,
                )
            )
        for cp in copies:
            cp.wait()

        def row(r, carry2):
            j = r * L
            acc0 = rows_v[j, 0:HALF]
            acc1 = rows_v[j, HALF:D]
            for l in range(1, L):
                acc0 = acc0 + rows_v[j + l, 0:HALF]
                acc1 = acc1 + rows_v[j + l, HALF:D]
            orow = g * BR + r
            out_v[orow, 0:HALF] = acc0 * INV_L
            out_v[orow, HALF:D] = acc1 * INV_L
            return carry2

        lax.fori_loop(0, BR, row, 0)
        return carry

    lax.fori_loop(0, NBLK, block, 0)
    pltpu.sync_copy(out_v, out_hbm.at[pl.ds(wid * BPW, BPW)])


def kernel(inputs, table):
    idx = inputs.reshape(-1).astype(jnp.int32)
    return _emb_lookup_mean(table, idx)


# speedup vs baseline: 2.9434x; 1.0678x over previous
import functools

import jax
import jax.numpy as jnp
from jax import lax
from jax.experimental import pallas as pl
from jax.experimental.pallas import tpu as pltpu
from jax.experimental.pallas import tpu_sc as plsc

NC, NS = 2, 16
NW = NC * NS
B, L, D = 16384, 50, 32
BPW = B // NW
BR = 16
NBLK = BPW // BR
IDX_PER_W = BPW * L
IDX_PER_BLK = BR * L
HALF = 16
INV_L = 1.0 / L

CHUNKS = [(off, 128) for off in range(0, 768, 128)] + [(768, 32)]

_mesh = plsc.VectorSubcoreMesh(
    core_axis_name="c", subcore_axis_name="s", num_cores=NC, num_subcores=NS
)


@functools.partial(
    pl.kernel,
    out_type=jax.ShapeDtypeStruct((B, D), jnp.float32),
    mesh=_mesh,
    scratch_types=[
        pltpu.VMEM((IDX_PER_W,), jnp.int32),
        pltpu.VMEM((2, IDX_PER_BLK, D), jnp.float32),
        pltpu.VMEM((BPW, D), jnp.float32),
        pltpu.SemaphoreType.DMA,
        pltpu.SemaphoreType.DMA,
    ],
    compiler_params=pltpu.CompilerParams(use_tc_tiling_on_sc=False),
)
def _emb_lookup_mean(table_hbm, idx_hbm, out_hbm, idx_v, rows_v, out_v,
                     sem0, sem1):
    sems = (sem0, sem1)
    wid = lax.axis_index("s") * NC + lax.axis_index("c")
    pltpu.sync_copy(idx_hbm.at[pl.ds(wid * IDX_PER_W, IDX_PER_W)], idx_v)

    def fire(p, blk):
        base = pl.multiple_of(blk * IDX_PER_BLK, 8)
        for off, n in CHUNKS:
            pltpu.async_copy(
                table_hbm.at[idx_v.at[pl.ds(base + off, n)]],
                rows_v.at[p, pl.ds(off, n)],
                sems[p],
            )

    def drain(p):
        for off, n in CHUNKS:
            pltpu.make_async_copy(
                table_hbm.at[idx_v.at[pl.ds(off, n)]],
                rows_v.at[p, pl.ds(off, n)],
                sems[p],
            ).wait()

    def accum(p, blk):
        def row(r, carry):
            j = r * L
            acc0 = rows_v[p, j, 0:HALF]
            acc1 = rows_v[p, j, HALF:D]
            for l in range(1, L):
                acc0 = acc0 + rows_v[p, j + l, 0:HALF]
                acc1 = acc1 + rows_v[p, j + l, HALF:D]
            orow = blk * BR + r
            out_v[orow, 0:HALF] = acc0 * INV_L
            out_v[orow, HALF:D] = acc1 * INV_L
            return carry

        lax.fori_loop(0, BR, row, 0)

    fire(0, 0)

    def body(g2, carry):
        ga = 2 * g2
        fire(1, ga + 1)
        drain(0)
        accum(0, ga)
        fire(0, ga + 2)
        drain(1)
        accum(1, ga + 1)
        return carry

    lax.fori_loop(0, NBLK // 2 - 1, body, 0)

    fire(1, NBLK - 1)
    drain(0)
    accum(0, NBLK - 2)
    drain(1)
    accum(1, NBLK - 1)

    pltpu.sync_copy(out_v, out_hbm.at[pl.ds(wid * BPW, BPW)])


def kernel(inputs, table):
    idx = inputs.reshape(-1).astype(jnp.int32)
    return _emb_lookup_mean(table, idx)


# speedup vs baseline: 2.9440x; 1.0002x over previous
import functools

import jax
import jax.numpy as jnp
from jax import lax
from jax.experimental import pallas as pl
from jax.experimental.pallas import tpu as pltpu
from jax.experimental.pallas import tpu_sc as plsc

NC, NS = 2, 16
NW = NC * NS
B, L, D = 16384, 50, 32
BPW = B // NW
BR = 16
NBLK = BPW // BR
IDX_PER_W = BPW * L
IDX_PER_BLK = BR * L
HALF = 16
INV_L = 1.0 / L

CHUNKS = [(0, IDX_PER_BLK)]

_mesh = plsc.VectorSubcoreMesh(
    core_axis_name="c", subcore_axis_name="s", num_cores=NC, num_subcores=NS
)


@functools.partial(
    pl.kernel,
    out_type=jax.ShapeDtypeStruct((B, D), jnp.float32),
    mesh=_mesh,
    scratch_types=[
        pltpu.VMEM((IDX_PER_W,), jnp.int32),
        pltpu.VMEM((2, IDX_PER_BLK, D), jnp.float32),
        pltpu.VMEM((BPW, D), jnp.float32),
        pltpu.SemaphoreType.DMA,
        pltpu.SemaphoreType.DMA,
    ],
    compiler_params=pltpu.CompilerParams(use_tc_tiling_on_sc=False),
)
def _emb_lookup_mean(table_hbm, idx_hbm, out_hbm, idx_v, rows_v, out_v,
                     sem0, sem1):
    sems = (sem0, sem1)
    wid = lax.axis_index("s") * NC + lax.axis_index("c")
    pltpu.sync_copy(idx_hbm.at[pl.ds(wid * IDX_PER_W, IDX_PER_W)], idx_v)

    def fire(p, blk):
        base = pl.multiple_of(blk * IDX_PER_BLK, 8)
        for off, n in CHUNKS:
            pltpu.async_copy(
                table_hbm.at[idx_v.at[pl.ds(base + off, n)]],
                rows_v.at[p, pl.ds(off, n)],
                sems[p],
            )

    def drain(p):
        for off, n in CHUNKS:
            pltpu.make_async_copy(
                table_hbm.at[idx_v.at[pl.ds(off, n)]],
                rows_v.at[p, pl.ds(off, n)],
                sems[p],
            ).wait()

    def accum(p, blk):
        def row(r, carry):
            j = r * L
            acc0 = rows_v[p, j, 0:HALF]
            acc1 = rows_v[p, j, HALF:D]
            for l in range(1, L):
                acc0 = acc0 + rows_v[p, j + l, 0:HALF]
                acc1 = acc1 + rows_v[p, j + l, HALF:D]
            orow = blk * BR + r
            out_v[orow, 0:HALF] = acc0 * INV_L
            out_v[orow, HALF:D] = acc1 * INV_L
            return carry

        lax.fori_loop(0, BR, row, 0)

    fire(0, 0)

    def body(g2, carry):
        ga = 2 * g2
        fire(1, ga + 1)
        drain(0)
        accum(0, ga)
        fire(0, ga + 2)
        drain(1)
        accum(1, ga + 1)
        return carry

    lax.fori_loop(0, NBLK // 2 - 1, body, 0)

    fire(1, NBLK - 1)
    drain(0)
    accum(0, NBLK - 2)
    drain(1)
    accum(1, NBLK - 1)

    pltpu.sync_copy(out_v, out_hbm.at[pl.ds(wid * BPW, BPW)])


def kernel(inputs, table):
    idx = inputs.reshape(-1).astype(jnp.int32)
    return _emb_lookup_mean(table, idx)
